# baseline (speedup 1.0000x reference)
import jax
import jax.numpy as jnp
from jax.experimental import pallas as pl
from jax.experimental.pallas import tpu as pltpu

N = 2048
NFEAT = 512
NHID = 64
DHID1 = 64
DEC_ROWS = 256


def _bf(v):
    return v.astype(jnp.bfloat16)


def _dot(a, b):
    return jax.lax.dot(a, b, preferred_element_type=jnp.float32)


NSPLIT = 8
ROWS = N // NSPLIT


def _gcn_body(*refs):
    adj_refs = refs[:NSPLIT]
    x_ref, w1_ref, w2_ref, out_ref = refs[NSPLIT:]
    slabs = [_bf(r[0, 0]) for r in adj_refs]
    p = _bf(_dot(_bf(x_ref[...]), _bf(w1_ref[0])))
    h = jnp.concatenate([jnp.maximum(_dot(s, p), 0.0) for s in slabs], axis=0)
    q = _bf(_dot(_bf(h), _bf(w2_ref[0])))
    for j, s in enumerate(slabs):
        out_ref[0, j * ROWS:(j + 1) * ROWS, :] = _dot(s, q)


def _gcn_call(adjs, x, w1, w2):
    adjs4 = adjs.reshape(3, NSPLIT, ROWS, N)

    def _slab_spec(j):
        return pl.BlockSpec((1, 1, ROWS, N), lambda v: (v, j, 0, 0))

    return pl.pallas_call(
        _gcn_body,
        grid=(3,),
        in_specs=[_slab_spec(j) for j in range(NSPLIT)] + [
            pl.BlockSpec((N, NFEAT), lambda v: (0, 0)),
            pl.BlockSpec((1, NFEAT, NHID), lambda v: (v, 0, 0)),
            pl.BlockSpec((1, NHID, NHID), lambda v: (v, 0, 0)),
        ],
        out_specs=pl.BlockSpec((1, N, NHID), lambda v: (v, 0, 0)),
        out_shape=jax.ShapeDtypeStruct((3, N, NHID), jnp.float32),
        compiler_params=pltpu.CompilerParams(
            vmem_limit_bytes=100 * 1024 * 1024,
        ),
    )(*([adjs4] * NSPLIT), x, w1, w2)


def _dsn_body(op_ref, oa_ref, on_ref, attw_ref,
              w1_ref, b1_ref, w2_ref, b2_ref, w3_ref, b3_ref,
              aw1_ref, ab1_ref, aw2_ref, ab2_ref, aw3_ref, ab3_ref,
              dec_ref, embed_ref, z_ref):
    embs = []
    for v in range(3):
        w1 = w1_ref[v]
        h = jnp.maximum(
            _dot(op_ref[v], w1[0 * NHID:1 * NHID])
            + _dot(oa_ref[v], w1[1 * NHID:2 * NHID])
            + _dot(on_ref[v], w1[2 * NHID:3 * NHID])
            + b1_ref[v:v + 1, :], 0.0)
        h = jnp.maximum(_dot(h, w2_ref[v]) + b2_ref[v:v + 1, :], 0.0)
        embs.append(_dot(h, w3_ref[v]) + b3_ref[v:v + 1, :])
    main, e1, e2 = embs
    aw = attw_ref[...]
    m = jnp.max(aw, axis=1, keepdims=True)
    ex = jnp.exp(aw - m)
    s = ex / jnp.sum(ex, axis=1, keepdims=True)
    s1 = e1 * s[:, 0:1]
    s2 = e2 * s[:, 1:2]
    g = jnp.maximum(
        _dot(s1, aw1_ref[0:DHID1]) + _dot(s2, aw1_ref[DHID1:2 * DHID1])
        + ab1_ref[...], 0.0)
    g = jnp.maximum(_dot(g, aw2_ref[...]) + ab2_ref[...], 0.0)
    sagg = _dot(g, aw3_ref[...]) + ab3_ref[...]
    embed_ref[:, 0:DHID1] = main
    embed_ref[:, DHID1:2 * DHID1] = sagg
    z_ref[...] = (_dot(main, dec_ref[0:DHID1])
                  + _dot(sagg, dec_ref[DHID1:2 * DHID1]))


def _dec_body(z_ref, embed_ref, out_ref):
    zz = _bf(z_ref[...])
    ee = _bf(embed_ref[...])
    logits = jax.lax.dot_general(
        zz, ee, dimension_numbers=(((1,), (1,)), ((), ())),
        preferred_element_type=jnp.float32)
    out_ref[...] = jax.nn.sigmoid(logits)


def kernel(x, adjs_pos, adjs_add, adjs_neg, attW, enc_W1, enc_W2,
           dsn_W1, dsn_b1, dsn_W2, dsn_b2, dsn_W3, dsn_b3,
           agg_W1, agg_b1, agg_W2, agg_b2, agg_W3, agg_b3, dec_W):
    outs = []
    for t, adjs in enumerate((adjs_pos, adjs_add, adjs_neg)):
        outs.append(_gcn_call(adjs, x, enc_W1[:, t], enc_W2[:, t]))
    o_pos, o_add, o_neg = outs

    embed, z = pl.pallas_call(
        _dsn_body,
        out_shape=(
            jax.ShapeDtypeStruct((N, 2 * DHID1), jnp.float32),
            jax.ShapeDtypeStruct((N, 2 * DHID1), jnp.float32),
        ),
    )(o_pos, o_add, o_neg, attW.reshape(1, 2),
      dsn_W1, dsn_b1, dsn_W2, dsn_b2, dsn_W3, dsn_b3,
      agg_W1, agg_b1.reshape(1, -1), agg_W2, agg_b2.reshape(1, -1),
      agg_W3, agg_b3.reshape(1, -1), dec_W)

    y = pl.pallas_call(
        _dec_body,
        grid=(N // DEC_ROWS,),
        in_specs=[
            pl.BlockSpec((DEC_ROWS, 2 * DHID1), lambda i: (i, 0)),
            pl.BlockSpec((N, 2 * DHID1), lambda i: (0, 0)),
        ],
        out_specs=pl.BlockSpec((DEC_ROWS, N), lambda i: (i, 0)),
        out_shape=jax.ShapeDtypeStruct((N, N), jnp.float32),
    )(z, embed)
    return y


# speedup vs baseline: 1.0487x; 1.0487x over previous
import jax
import jax.numpy as jnp
from jax.experimental import pallas as pl
from jax.experimental.pallas import tpu as pltpu

N = 2048
NFEAT = 512
NHID = 64
DHID1 = 64
DEC_ROWS = 256


def _bf(v):
    return v.astype(jnp.bfloat16)


def _dot(a, b):
    return jax.lax.dot(a, b, preferred_element_type=jnp.float32)


def _bdot(a, b):
    return jax.lax.dot(_bf(a), _bf(b), preferred_element_type=jnp.float32)


NSPLIT = 4
ROWS = N // NSPLIT


def _gcn_body(*refs):
    adj_refs = refs[:NSPLIT]
    x_ref, w1_ref, w2_ref, out_ref = refs[NSPLIT:]
    slabs = [_bf(r[0, 0]) for r in adj_refs]
    p = _bf(_dot(_bf(x_ref[...]), _bf(w1_ref[0])))
    h = jnp.concatenate([jnp.maximum(_dot(s, p), 0.0) for s in slabs], axis=0)
    q = _bf(_dot(_bf(h), _bf(w2_ref[0])))
    for j, s in enumerate(slabs):
        out_ref[0, j * ROWS:(j + 1) * ROWS, :] = _dot(s, q)


def _gcn_call(adjs, x, w1, w2):
    adjs4 = adjs.reshape(3, NSPLIT, ROWS, N)

    def _slab_spec(j):
        return pl.BlockSpec((1, 1, ROWS, N), lambda v: (v, j, 0, 0))

    return pl.pallas_call(
        _gcn_body,
        grid=(3,),
        in_specs=[_slab_spec(j) for j in range(NSPLIT)] + [
            pl.BlockSpec((N, NFEAT), lambda v: (0, 0)),
            pl.BlockSpec((1, NFEAT, NHID), lambda v: (v, 0, 0)),
            pl.BlockSpec((1, NHID, NHID), lambda v: (v, 0, 0)),
        ],
        out_specs=pl.BlockSpec((1, N, NHID), lambda v: (v, 0, 0)),
        out_shape=jax.ShapeDtypeStruct((3, N, NHID), jnp.float32),
        compiler_params=pltpu.CompilerParams(
            vmem_limit_bytes=100 * 1024 * 1024,
        ),
    )(*([adjs4] * NSPLIT), x, w1, w2)


def _dsn_body(op_ref, oa_ref, on_ref, attw_ref,
              w1_ref, b1_ref, w2_ref, b2_ref, w3_ref, b3_ref,
              aw1_ref, ab1_ref, aw2_ref, ab2_ref, aw3_ref, ab3_ref,
              dec_ref, embed_ref, z_ref):
    embs = []
    for v in range(3):
        w1 = w1_ref[v]
        h = jnp.maximum(
            _bdot(op_ref[v], w1[0 * NHID:1 * NHID])
            + _bdot(oa_ref[v], w1[1 * NHID:2 * NHID])
            + _bdot(on_ref[v], w1[2 * NHID:3 * NHID])
            + b1_ref[v:v + 1, :], 0.0)
        h = jnp.maximum(_bdot(h, w2_ref[v]) + b2_ref[v:v + 1, :], 0.0)
        embs.append(_bdot(h, w3_ref[v]) + b3_ref[v:v + 1, :])
    main, e1, e2 = embs
    aw = attw_ref[...]
    m = jnp.max(aw, axis=1, keepdims=True)
    ex = jnp.exp(aw - m)
    s = ex / jnp.sum(ex, axis=1, keepdims=True)
    s1 = e1 * s[:, 0:1]
    s2 = e2 * s[:, 1:2]
    g = jnp.maximum(
        _bdot(s1, aw1_ref[0:DHID1]) + _bdot(s2, aw1_ref[DHID1:2 * DHID1])
        + ab1_ref[...], 0.0)
    g = jnp.maximum(_bdot(g, aw2_ref[...]) + ab2_ref[...], 0.0)
    sagg = _bdot(g, aw3_ref[...]) + ab3_ref[...]
    embed_ref[:, 0:DHID1] = main
    embed_ref[:, DHID1:2 * DHID1] = sagg
    z_ref[...] = (_bdot(main, dec_ref[0:DHID1])
                  + _bdot(sagg, dec_ref[DHID1:2 * DHID1]))


def _dec_body(z_ref, embed_ref, out_ref):
    zz = _bf(z_ref[...])
    ee = _bf(embed_ref[...])
    logits = jax.lax.dot_general(
        zz, ee, dimension_numbers=(((1,), (1,)), ((), ())),
        preferred_element_type=jnp.float32)
    out_ref[...] = jax.nn.sigmoid(logits)


def kernel(x, adjs_pos, adjs_add, adjs_neg, attW, enc_W1, enc_W2,
           dsn_W1, dsn_b1, dsn_W2, dsn_b2, dsn_W3, dsn_b3,
           agg_W1, agg_b1, agg_W2, agg_b2, agg_W3, agg_b3, dec_W):
    outs = []
    for t, adjs in enumerate((adjs_pos, adjs_add, adjs_neg)):
        outs.append(_gcn_call(adjs, x, enc_W1[:, t], enc_W2[:, t]))
    o_pos, o_add, o_neg = outs

    embed, z = pl.pallas_call(
        _dsn_body,
        out_shape=(
            jax.ShapeDtypeStruct((N, 2 * DHID1), jnp.float32),
            jax.ShapeDtypeStruct((N, 2 * DHID1), jnp.float32),
        ),
    )(o_pos, o_add, o_neg, attW.reshape(1, 2),
      dsn_W1, dsn_b1, dsn_W2, dsn_b2, dsn_W3, dsn_b3,
      agg_W1, agg_b1.reshape(1, -1), agg_W2, agg_b2.reshape(1, -1),
      agg_W3, agg_b3.reshape(1, -1), dec_W)

    y = pl.pallas_call(
        _dec_body,
        grid=(N // DEC_ROWS,),
        in_specs=[
            pl.BlockSpec((DEC_ROWS, 2 * DHID1), lambda i: (i, 0)),
            pl.BlockSpec((N, 2 * DHID1), lambda i: (0, 0)),
        ],
        out_specs=pl.BlockSpec((DEC_ROWS, N), lambda i: (i, 0)),
        out_shape=jax.ShapeDtypeStruct((N, N), jnp.float32),
    )(z, embed)
    return y


# speedup vs baseline: 1.1955x; 1.1400x over previous
import jax
import jax.numpy as jnp
from jax.experimental import pallas as pl
from jax.experimental.pallas import tpu as pltpu

N = 2048
NFEAT = 512
NHID = 64
DHID1 = 64
DEC_ROWS = 256


def _bf(v):
    return v.astype(jnp.bfloat16)


def _dot(a, b):
    return jax.lax.dot(a, b, preferred_element_type=jnp.float32)


def _bdot(a, b):
    return jax.lax.dot(_bf(a), _bf(b), preferred_element_type=jnp.float32)


NSLAB = 8
SROWS = N // NSLAB


def _gcn_body(ap_ref, aa_ref, an_ref, x_ref, w1_ref, w2_ref, out_ref,
              buf_ref, sem_ref):
    g = pl.program_id(0)

    def _start(step, slot):
        tt = step // 3
        vv = step % 3
        for k, ar in enumerate((ap_ref, aa_ref, an_ref)):
            @pl.when(tt == k)
            def _():
                for j in range(NSLAB):
                    pltpu.make_async_copy(
                        ar.at[vv, pl.ds(j * SROWS, SROWS), :],
                        buf_ref.at[slot, j],
                        sem_ref.at[slot, j]).start()

    @pl.when(g == 0)
    def _():
        _start(g, 0)

    @pl.when(g < 8)
    def _():
        _start(g + 1, jax.lax.rem(g + 1, 2))

    slot = jax.lax.rem(g, 2)
    for j in range(NSLAB):
        pltpu.make_async_copy(
            ap_ref.at[0, pl.ds(j * SROWS, SROWS), :],
            buf_ref.at[slot, j],
            sem_ref.at[slot, j]).wait()

    p = _bf(_dot(_bf(x_ref[...]), _bf(w1_ref[0, 0])))
    slabs = [_bf(buf_ref[slot, j]) for j in range(NSLAB)]
    h = jnp.concatenate([jnp.maximum(_dot(s, p), 0.0) for s in slabs], axis=0)
    q = _bf(_dot(_bf(h), _bf(w2_ref[0, 0])))
    for j in range(NSLAB):
        out_ref[0, 0, j * SROWS:(j + 1) * SROWS, :] = _dot(slabs[j], q)


def _gcn_call(adjs_pos, adjs_add, adjs_neg, x, enc_W1, enc_W2):
    return pl.pallas_call(
        _gcn_body,
        grid=(9,),
        in_specs=[
            pl.BlockSpec(memory_space=pl.ANY),
            pl.BlockSpec(memory_space=pl.ANY),
            pl.BlockSpec(memory_space=pl.ANY),
            pl.BlockSpec((N, NFEAT), lambda g: (0, 0)),
            pl.BlockSpec((1, 1, NFEAT, NHID), lambda g: (g % 3, g // 3, 0, 0)),
            pl.BlockSpec((1, 1, NHID, NHID), lambda g: (g % 3, g // 3, 0, 0)),
        ],
        out_specs=pl.BlockSpec((1, 1, N, NHID), lambda g: (g // 3, g % 3, 0, 0)),
        out_shape=jax.ShapeDtypeStruct((3, 3, N, NHID), jnp.float32),
        scratch_shapes=[
            pltpu.VMEM((2, NSLAB, SROWS, N), jnp.float32),
            pltpu.SemaphoreType.DMA((2, NSLAB)),
        ],
        compiler_params=pltpu.CompilerParams(
            vmem_limit_bytes=100 * 1024 * 1024,
        ),
    )(adjs_pos, adjs_add, adjs_neg, x, enc_W1, enc_W2)


def _dsn_body(o_ref, attw_ref,
              w1_ref, b1_ref, w2_ref, b2_ref, w3_ref, b3_ref,
              aw1_ref, ab1_ref, aw2_ref, ab2_ref, aw3_ref, ab3_ref,
              dec_ref, embed_ref, z_ref):
    embs = []
    for v in range(3):
        w1 = w1_ref[v]
        h = jnp.maximum(
            _bdot(o_ref[0, v], w1[0 * NHID:1 * NHID])
            + _bdot(o_ref[1, v], w1[1 * NHID:2 * NHID])
            + _bdot(o_ref[2, v], w1[2 * NHID:3 * NHID])
            + b1_ref[v:v + 1, :], 0.0)
        h = jnp.maximum(_bdot(h, w2_ref[v]) + b2_ref[v:v + 1, :], 0.0)
        embs.append(_bdot(h, w3_ref[v]) + b3_ref[v:v + 1, :])
    main, e1, e2 = embs
    aw = attw_ref[...]
    m = jnp.max(aw, axis=1, keepdims=True)
    ex = jnp.exp(aw - m)
    s = ex / jnp.sum(ex, axis=1, keepdims=True)
    s1 = e1 * s[:, 0:1]
    s2 = e2 * s[:, 1:2]
    g = jnp.maximum(
        _bdot(s1, aw1_ref[0:DHID1]) + _bdot(s2, aw1_ref[DHID1:2 * DHID1])
        + ab1_ref[...], 0.0)
    g = jnp.maximum(_bdot(g, aw2_ref[...]) + ab2_ref[...], 0.0)
    sagg = _bdot(g, aw3_ref[...]) + ab3_ref[...]
    embed_ref[:, 0:DHID1] = main
    embed_ref[:, DHID1:2 * DHID1] = sagg
    z_ref[...] = (_bdot(main, dec_ref[0:DHID1])
                  + _bdot(sagg, dec_ref[DHID1:2 * DHID1]))


def _dec_body(z_ref, embed_ref, out_ref):
    zz = _bf(z_ref[...])
    ee = _bf(embed_ref[...])
    logits = jax.lax.dot_general(
        zz, ee, dimension_numbers=(((1,), (1,)), ((), ())),
        preferred_element_type=jnp.float32)
    out_ref[...] = jax.nn.sigmoid(logits)


def kernel(x, adjs_pos, adjs_add, adjs_neg, attW, enc_W1, enc_W2,
           dsn_W1, dsn_b1, dsn_W2, dsn_b2, dsn_W3, dsn_b3,
           agg_W1, agg_b1, agg_W2, agg_b2, agg_W3, agg_b3, dec_W):
    o = _gcn_call(adjs_pos, adjs_add, adjs_neg, x, enc_W1, enc_W2)

    embed, z = pl.pallas_call(
        _dsn_body,
        out_shape=(
            jax.ShapeDtypeStruct((N, 2 * DHID1), jnp.float32),
            jax.ShapeDtypeStruct((N, 2 * DHID1), jnp.float32),
        ),
    )(o, attW.reshape(1, 2),
      dsn_W1, dsn_b1, dsn_W2, dsn_b2, dsn_W3, dsn_b3,
      agg_W1, agg_b1.reshape(1, -1), agg_W2, agg_b2.reshape(1, -1),
      agg_W3, agg_b3.reshape(1, -1), dec_W)

    y = pl.pallas_call(
        _dec_body,
        grid=(N // DEC_ROWS,),
        in_specs=[
            pl.BlockSpec((DEC_ROWS, 2 * DHID1), lambda i: (i, 0)),
            pl.BlockSpec((N, 2 * DHID1), lambda i: (0, 0)),
        ],
        out_specs=pl.BlockSpec((DEC_ROWS, N), lambda i: (i, 0)),
        out_shape=jax.ShapeDtypeStruct((N, N), jnp.float32),
    )(z, embed)
    return y
